# baseline (speedup 1.0000x reference)
import functools

import jax
import jax.numpy as jnp
from jax import lax
from jax.experimental import pallas as pl
from jax.experimental.pallas import tpu as pltpu
from jax.experimental.pallas import tpu_sc as plsc

_NC, _NS, _L = 2, 16, 16
_NW = _NC * _NS


def _tc_body(t_ref, p_ref, e_ref, out_ref, acc0_ref, acc2_ref):
    i = pl.program_id(0)
    n = pl.num_programs(0)

    @pl.when(i == 0)
    def _init():
        acc0_ref[...] = jnp.zeros_like(acc0_ref)
        acc2_ref[...] = jnp.zeros_like(acc2_ref)

    d = t_ref[...] - p_ref[...]
    acc0_ref[...] += jnp.sum((d * d).reshape(-1, 8, 128), axis=0)
    e = jnp.maximum(e_ref[...], 0.0)
    acc2_ref[...] += jnp.sum((e * e).reshape(-1, 8, 128), axis=0)

    @pl.when(i == n - 1)
    def _fin():
        out_ref[0, 0] = jnp.sum(acc0_ref[...])
        out_ref[0, 1] = jnp.sum(acc2_ref[...])


def _tc_sums(target, prediction, E):
    N, D = target.shape
    BLK = 4000
    rowblk = pl.BlockSpec((BLK, D), lambda i: (i, 0))
    return pl.pallas_call(
        _tc_body,
        grid=(N // BLK,),
        in_specs=[rowblk, rowblk, rowblk],
        out_specs=pl.BlockSpec(memory_space=pltpu.SMEM),
        out_shape=jax.ShapeDtypeStruct((1, 2), jnp.float32),
        scratch_shapes=[pltpu.VMEM((8, 128), jnp.float32)] * 2,
        compiler_params=pltpu.CompilerParams(
            dimension_semantics=("arbitrary",)),
    )(target, prediction, E)


_CH = 4096
_NFULL = 100000 // _CH
_TAIL = 100000 - _NFULL * _CH
_UNROLL = 4


def _reduce_rows(buf_ref, slot, minor, accs):
    iters = minor // (_L * _UNROLL)
    for r in range(8):
        def body(i, accs, r=r):
            base = i * (_L * _UNROLL)
            new = []
            for u in range(_UNROLL):
                x = buf_ref[slot, r, pl.ds(base + u * _L, _L)]
                new.append(accs[u] + x * jnp.maximum(x, 0.0))
            return tuple(new)

        accs = lax.fori_loop(0, iters, body, accs)
    return accs


def _sc_w_kernel(w_hbm, out_hbm, buf, tbuf, accv, sem0, sem1, tsem):
    wid = lax.axis_index("s") * _NC + lax.axis_index("c")
    accv[...] = jnp.zeros((_L,), jnp.float32)

    @pl.when(wid < 16)
    def _active():
        r0 = wid * 8
        sems = (sem0, sem1)

        def start(t, slot):
            return pltpu.async_copy(
                w_hbm.at[pl.ds(r0, 8), pl.ds(t * _CH, _CH)],
                buf.at[slot], sems[slot])

        handles = {0: start(0, 0)}
        tail_handle = pltpu.async_copy(
            w_hbm.at[pl.ds(r0, 8), pl.ds(_NFULL * _CH, _TAIL)], tbuf, tsem)
        zero = jnp.zeros((_L,), jnp.float32)
        accs = (zero,) * _UNROLL
        for t in range(_NFULL):
            slot = t % 2
            if t + 1 < _NFULL:
                handles[t + 1] = start(t + 1, (t + 1) % 2)
            handles[t].wait()
            accs = _reduce_rows(buf, slot, _CH, accs)
        tail_handle.wait()
        titers = _TAIL // (_L * _UNROLL)
        for r in range(8):
            def tbody(i, accs, r=r):
                base = i * _L
                x = tbuf[r, pl.ds(base, _L)]
                return (accs[0] + x * jnp.maximum(x, 0.0),) + accs[1:]

            accs = lax.fori_loop(0, _TAIL // _L, tbody, accs)
        accv[...] = accs[0] + accs[1] + accs[2] + accs[3]

    pltpu.sync_copy(accv, out_hbm.at[wid])


def _sc_w_partials(W):
    kern = functools.partial(
        pl.kernel,
        mesh=plsc.VectorSubcoreMesh(core_axis_name="c", subcore_axis_name="s"),
        out_type=jax.ShapeDtypeStruct((_NW, _L), jnp.float32),
        scratch_types=[
            pltpu.VMEM((2, 8, _CH), jnp.float32),
            pltpu.VMEM((8, _TAIL), jnp.float32),
            pltpu.VMEM((_L,), jnp.float32),
            pltpu.SemaphoreType.DMA,
            pltpu.SemaphoreType.DMA,
            pltpu.SemaphoreType.DMA,
        ],
    )(_sc_w_kernel)
    return kern(W)


def kernel(target, prediction, reg, batch, W, E, Sw, Se):
    w_part = _sc_w_partials(W)
    tc = _tc_sums(target, prediction, E)
    s0, s2 = tc[0, 0], tc[0, 1]
    s1 = jnp.sum(w_part)
    return 0.5 * jnp.sqrt(s0) + reg[2] * (jnp.sqrt(s1) + jnp.sqrt(s2))


# speedup vs baseline: 1.0026x; 1.0026x over previous
import functools

import jax
import jax.numpy as jnp
from jax import lax
from jax.experimental import pallas as pl
from jax.experimental.pallas import tpu as pltpu
from jax.experimental.pallas import tpu_sc as plsc

_NC, _NS, _L = 2, 16, 16
_NW = _NC * _NS


def _tc_body(t_ref, p_ref, e_ref, out_ref, acc0_ref, acc2_ref):
    i = pl.program_id(0)
    n = pl.num_programs(0)

    @pl.when(i == 0)
    def _init():
        acc0_ref[...] = jnp.zeros_like(acc0_ref)
        acc2_ref[...] = jnp.zeros_like(acc2_ref)

    d = t_ref[...] - p_ref[...]
    acc0_ref[...] += jnp.sum((d * d).reshape(-1, 8, 128), axis=0)
    e = jnp.maximum(e_ref[...], 0.0)
    acc2_ref[...] += jnp.sum((e * e).reshape(-1, 8, 128), axis=0)

    @pl.when(i == n - 1)
    def _fin():
        out_ref[0, 0] = jnp.sum(acc0_ref[...])
        out_ref[0, 1] = jnp.sum(acc2_ref[...])


def _tc_sums(target, prediction, E):
    N, D = target.shape
    BLK = 4000
    rowblk = pl.BlockSpec((BLK, D), lambda i: (i, 0))
    return pl.pallas_call(
        _tc_body,
        grid=(N // BLK,),
        in_specs=[rowblk, rowblk, rowblk],
        out_specs=pl.BlockSpec(memory_space=pltpu.SMEM),
        out_shape=jax.ShapeDtypeStruct((1, 2), jnp.float32),
        scratch_shapes=[pltpu.VMEM((8, 128), jnp.float32)] * 2,
        compiler_params=pltpu.CompilerParams(
            dimension_semantics=("arbitrary",)),
    )(target, prediction, E)


_CH = 4096
_NFULL = 100000 // _CH
_TAIL = 100000 - _NFULL * _CH
_UNROLL = 4


def _reduce_rows(buf_ref, slot, minor, accs):
    iters = minor // (_L * _UNROLL)
    for r in range(8):
        def body(i, accs, r=r):
            base = i * (_L * _UNROLL)
            new = []
            for u in range(_UNROLL):
                x = buf_ref[slot, r, pl.ds(base + u * _L, _L)]
                new.append(accs[u] + x * jnp.maximum(x, 0.0))
            return tuple(new)

        accs = lax.fori_loop(0, iters, body, accs)
    return accs


def _sc_w_kernel(w_hbm, out_hbm, buf, tbuf, accv, sem0, sem1, tsem):
    wid = lax.axis_index("s") * _NC + lax.axis_index("c")
    accv[...] = jnp.zeros((_L,), jnp.float32)

    @pl.when(wid < 16)
    def _active():
        r0 = wid * 8
        sems = (sem0, sem1)

        def start(t, slot):
            return pltpu.async_copy(
                w_hbm.at[pl.ds(r0, 8), pl.ds(t * _CH, _CH)],
                buf.at[slot], sems[slot])

        handles = {0: start(0, 0)}
        tail_handle = pltpu.async_copy(
            w_hbm.at[pl.ds(r0, 8), pl.ds(_NFULL * _CH, _TAIL)], tbuf, tsem)
        zero = jnp.zeros((_L,), jnp.float32)
        accs = (zero,) * _UNROLL
        for t in range(_NFULL):
            slot = t % 2
            if t + 1 < _NFULL:
                handles[t + 1] = start(t + 1, (t + 1) % 2)
            handles[t].wait()
            accs = _reduce_rows(buf, slot, _CH, accs)
        tail_handle.wait()
        titers = _TAIL // (_L * _UNROLL)
        for r in range(8):
            def tbody(i, accs, r=r):
                base = i * _L
                x = tbuf[r, pl.ds(base, _L)]
                return (accs[0] + x * jnp.maximum(x, 0.0),) + accs[1:]

            accs = lax.fori_loop(0, _TAIL // _L, tbody, accs)
        accv[...] = accs[0] + accs[1] + accs[2] + accs[3]

    pltpu.sync_copy(accv, out_hbm.at[wid])


def _sc_w_partials(W):
    kern = functools.partial(
        pl.kernel,
        mesh=plsc.VectorSubcoreMesh(core_axis_name="c", subcore_axis_name="s"),
        out_type=jax.ShapeDtypeStruct((_NW, _L), jnp.float32),
        scratch_types=[
            pltpu.VMEM((2, 8, _CH), jnp.float32),
            pltpu.VMEM((8, _TAIL), jnp.float32),
            pltpu.VMEM((_L,), jnp.float32),
            pltpu.SemaphoreType.DMA,
            pltpu.SemaphoreType.DMA,
            pltpu.SemaphoreType.DMA,
        ],
        compiler_params=pltpu.CompilerParams(use_tc_tiling_on_sc=True),
    )(_sc_w_kernel)
    return kern(W)


def kernel(target, prediction, reg, batch, W, E, Sw, Se):
    w_part = _sc_w_partials(W)
    tc = _tc_sums(target, prediction, E)
    s0, s2 = tc[0, 0], tc[0, 1]
    s1 = jnp.sum(w_part)
    return 0.5 * jnp.sqrt(s0) + reg[2] * (jnp.sqrt(s1) + jnp.sqrt(s2))


# speedup vs baseline: 2.2119x; 2.2062x over previous
import jax
import jax.numpy as jnp
from jax.experimental import pallas as pl
from jax.experimental.pallas import tpu as pltpu


def _loss_body(reg_ref, t_ref, p_ref, w_ref, e_ref, out_ref,
               acc0_ref, acc1_ref, acc2_ref):
    i = pl.program_id(0)
    n = pl.num_programs(0)

    @pl.when(i == 0)
    def _init():
        acc0_ref[...] = jnp.zeros_like(acc0_ref)
        acc1_ref[...] = jnp.zeros_like(acc1_ref)
        acc2_ref[...] = jnp.zeros_like(acc2_ref)

    d = t_ref[...] - p_ref[...]
    acc0_ref[...] += jnp.sum((d * d).reshape(-1, 8, 128), axis=0)
    w = w_ref[...]
    acc1_ref[...] += jnp.sum((w * jnp.maximum(w, 0.0)).reshape(-1, 8, 128),
                             axis=0)
    e = e_ref[...]
    acc2_ref[...] += jnp.sum((e * jnp.maximum(e, 0.0)).reshape(-1, 8, 128),
                             axis=0)

    @pl.when(i == n - 1)
    def _fin():
        out_ref[0, 0] = (0.5 * jnp.sqrt(jnp.sum(acc0_ref[...]))
                         + reg_ref[2] * (jnp.sqrt(jnp.sum(acc1_ref[...]))
                                         + jnp.sqrt(jnp.sum(acc2_ref[...]))))


def kernel(target, prediction, reg, batch, W, E, Sw, Se):
    N, D = target.shape
    Wt = W.T
    BLK = 4000
    grid = N // BLK

    rowblk = pl.BlockSpec((BLK, D), lambda i: (i, 0))
    out = pl.pallas_call(
        _loss_body,
        grid=(grid,),
        in_specs=[
            pl.BlockSpec(memory_space=pltpu.SMEM),
            rowblk, rowblk, rowblk, rowblk,
        ],
        out_specs=pl.BlockSpec(memory_space=pltpu.SMEM),
        out_shape=jax.ShapeDtypeStruct((1, 1), jnp.float32),
        scratch_shapes=[pltpu.VMEM((8, 128), jnp.float32)] * 3,
        compiler_params=pltpu.CompilerParams(
            dimension_semantics=("arbitrary",)),
    )(reg, target, prediction, Wt, E)
    return out[0, 0]
